# baseline (speedup 1.0000x reference)
import functools

import jax
import jax.numpy as jnp
from jax import lax
from jax.experimental import pallas as pl
from jax.experimental.pallas import tpu as pltpu
from jax.experimental.pallas import tpu_sc as plsc

DM = 1024
DF = 2048
NE = 8
TOPK = 2
T = 8192
A = T * TOPK
BLK = 512
NG = A // BLK + NE
S = NG * BLK
TB = 512


NTB = T // TB


def _route_body(x_ref, wg_ref, bg_ref,
                w0_ref, w1_ref, p0_ref, p1_ref, be_ref,
                e0s, e1s, w0s, w1s, cnt, cnt2):
    p = pl.program_id(0)
    g = pl.program_id(1)
    cols = lax.broadcasted_iota(jnp.int32, (TB, NE), 1)

    @pl.when(p == 0)
    def _phase0():
        logits = jnp.dot(x_ref[...], wg_ref[...],
                         preferred_element_type=jnp.float32) + bg_ref[0]
        i0 = jnp.argmax(logits, axis=1).astype(jnp.int32)
        m0 = jnp.max(logits, axis=1)
        masked = jnp.where(cols == i0[:, None], -jnp.inf, logits)
        i1 = jnp.argmax(masked, axis=1).astype(jnp.int32)
        m1 = jnp.max(masked, axis=1)
        w0 = jax.nn.sigmoid(m0 - m1)
        e0s[g, :] = i0
        e1s[g, :] = i1
        w0s[g, :] = w0
        w1s[g, :] = 1.0 - w0
        oh = ((cols == i0[:, None]).astype(jnp.float32)
              + (cols == i1[:, None]).astype(jnp.float32))
        colsum = jnp.sum(oh, axis=0, keepdims=True)
        prev = jnp.where(g == 0, jnp.zeros_like(cnt[...]), cnt[...])
        cnt[...] = prev + colsum

    @pl.when(p == 1)
    def _phase1():
        tot_i = cnt[...].astype(jnp.int32)
        bpe = (tot_i + BLK - 1) >> 9
        iu = lax.broadcasted_iota(jnp.int32, (NE, NE), 0)
        ju = lax.broadcasted_iota(jnp.int32, (NE, NE), 1)
        um = (iu < ju).astype(jnp.float32)
        bf = jnp.dot(bpe.astype(jnp.float32), um,
                     preferred_element_type=jnp.float32)
        prev2 = jnp.where(g == 0, jnp.zeros_like(cnt2[...]), cnt2[...])
        i0 = e0s[g, :]
        i1 = e1s[g, :]
        oh0 = (cols == i0[:, None]).astype(jnp.float32)
        oh1 = (cols == i1[:, None]).astype(jnp.float32)
        ohs = oh0 + oh1
        ri = lax.broadcasted_iota(jnp.int32, (TB, TB), 0)
        ci = lax.broadcasted_iota(jnp.int32, (TB, TB), 1)
        stril = (ci < ri).astype(jnp.bfloat16)
        cumbt = jnp.dot(stril, ohs.astype(jnp.bfloat16),
                        preferred_element_type=jnp.float32)
        base = prev2 + cumbt
        r0 = jnp.sum(base * oh0, axis=1).astype(jnp.int32)
        r1 = jnp.sum(base * oh1, axis=1).astype(jnp.int32)
        bfb = jnp.broadcast_to(bf, (TB, NE))
        bf0 = jnp.sum(bfb * oh0, axis=1).astype(jnp.int32)
        bf1 = jnp.sum(bfb * oh1, axis=1).astype(jnp.int32)
        p0_ref[...] = ((bf0 + (r0 >> 9)) << 9) + (r0 & (BLK - 1))
        p1_ref[...] = ((bf1 + (r1 >> 9)) << 9) + (r1 & (BLK - 1))
        w0_ref[...] = w0s[g, :]
        w1_ref[...] = w1s[g, :]
        cnt2[...] = prev2 + jnp.sum(ohs, axis=0, keepdims=True)
        bidx = lax.broadcasted_iota(jnp.int32, (NG, NE), 0)
        bfg = jnp.broadcast_to(bf, (NG, NE)).astype(jnp.int32)
        be_ref[...] = jnp.clip(
            jnp.sum((bidx >= bfg).astype(jnp.int32), axis=1) - 1, 0, NE - 1)


def _route(xf, Wg, bg):
    return pl.pallas_call(
        _route_body,
        grid=(2, NTB),
        in_specs=[
            pl.BlockSpec((TB, DM), lambda p, g: (g * (1 - p), 0)),
            pl.BlockSpec((DM, NE), lambda p, g: (0, 0)),
            pl.BlockSpec((1, NE), lambda p, g: (0, 0)),
        ],
        out_specs=[
            pl.BlockSpec((TB,), lambda p, g: (g,)),
            pl.BlockSpec((TB,), lambda p, g: (g,)),
            pl.BlockSpec((TB,), lambda p, g: (g,)),
            pl.BlockSpec((TB,), lambda p, g: (g,)),
            pl.BlockSpec((NG,), lambda p, g: (0,)),
        ],
        out_shape=[
            jax.ShapeDtypeStruct((T,), jnp.float32),
            jax.ShapeDtypeStruct((T,), jnp.float32),
            jax.ShapeDtypeStruct((T,), jnp.int32),
            jax.ShapeDtypeStruct((T,), jnp.int32),
            jax.ShapeDtypeStruct((NG,), jnp.int32),
        ],
        scratch_shapes=[
            pltpu.VMEM((NTB, TB), jnp.int32),
            pltpu.VMEM((NTB, TB), jnp.int32),
            pltpu.VMEM((NTB, TB), jnp.float32),
            pltpu.VMEM((NTB, TB), jnp.float32),
            pltpu.VMEM((1, NE), jnp.float32),
            pltpu.VMEM((1, NE), jnp.float32),
        ],
    )(xf, Wg, bg.reshape(1, NE))


NF = 2
FH = DF // NF


def _ffn_body(be_ref, xg_ref, wg_ref, bg_ref, wu_ref, bu_ref, wd_ref, bd_ref,
              ws_ref, og_ref):
    f = pl.program_id(1)
    xb = xg_ref[...]
    g = jnp.dot(xb, wg_ref[0], precision=lax.Precision.DEFAULT,
                preferred_element_type=jnp.float32) + bg_ref[0]
    u = jnp.dot(xb, wu_ref[0], precision=lax.Precision.DEFAULT,
                preferred_element_type=jnp.float32) + bu_ref[0]
    t = g * u
    h = t * jax.nn.sigmoid(t)
    o = jnp.dot(h, wd_ref[0], precision=lax.Precision.DEFAULT,
                preferred_element_type=jnp.float32)

    @pl.when(f == 0)
    def _():
        og_ref[...] = o

    @pl.when(f == NF - 1)
    def _():
        acc = o + bd_ref[0] if NF == 1 else og_ref[...] + o + bd_ref[0]
        og_ref[...] = acc * ws_ref[...]


def _ffn(xg, W_gate, b_gate, W_up, b_up, W_down, b_down, w_slot, block_expert):
    grid_spec = pltpu.PrefetchScalarGridSpec(
        num_scalar_prefetch=1,
        grid=(NG, NF),
        in_specs=[
            pl.BlockSpec((BLK, DM), lambda g, f, be: (g, 0)),
            pl.BlockSpec((1, DM, FH), lambda g, f, be: (be[g], 0, f)),
            pl.BlockSpec((1, 1, FH), lambda g, f, be: (be[g], 0, f)),
            pl.BlockSpec((1, DM, FH), lambda g, f, be: (be[g], 0, f)),
            pl.BlockSpec((1, 1, FH), lambda g, f, be: (be[g], 0, f)),
            pl.BlockSpec((1, FH, DM), lambda g, f, be: (be[g], f, 0)),
            pl.BlockSpec((1, 1, DM), lambda g, f, be: (be[g], 0, 0)),
            pl.BlockSpec((BLK, 1), lambda g, f, be: (g, 0)),
        ],
        out_specs=pl.BlockSpec((BLK, DM), lambda g, f, be: (g, 0)),
    )
    return pl.pallas_call(
        _ffn_body,
        grid_spec=grid_spec,
        out_shape=jax.ShapeDtypeStruct((S, DM), jnp.float32),
    )(block_expert, xg, W_gate, b_gate.reshape(NE, 1, DF), W_up,
      b_up.reshape(NE, 1, DF), W_down, b_down.reshape(NE, 1, DM),
      w_slot.reshape(S, 1))


NW = 32
TPW = T // NW
DCH = 64
CCH = 32

_MESH = plsc.VectorSubcoreMesh(core_axis_name="c", subcore_axis_name="s")


@functools.partial(
    pl.kernel, mesh=_MESH,
    out_type=[jax.ShapeDtypeStruct((S, DM), jnp.float32),
              jax.ShapeDtypeStruct((S,), jnp.float32)],
    scratch_types=[
        pltpu.VMEM((DCH, DM), jnp.float32),
        pltpu.VMEM((DCH,), jnp.int32),
        pltpu.VMEM((DCH,), jnp.int32),
        pltpu.VMEM((DCH,), jnp.float32),
        pltpu.VMEM((DCH,), jnp.float32),
        pltpu.SemaphoreType.DMA,
        pltpu.SemaphoreType.DMA,
        pltpu.SemaphoreType.DMA,
        pltpu.SemaphoreType.DMA,
    ],
)
def _sc_dispatch(x_hbm, p0_hbm, p1_hbm, w0_hbm, w1_hbm, xg_hbm, ws_hbm,
                 xbuf, p0v, p1v, w0v, w1v, s1, s2, s3, s4):
    wid = lax.axis_index("s") * 2 + lax.axis_index("c")
    base = wid * TPW

    def body(i, carry):
        off = base + i * DCH
        pltpu.sync_copy(x_hbm.at[pl.ds(off, DCH)], xbuf)
        pltpu.sync_copy(p0_hbm.at[pl.ds(off, DCH)], p0v)
        pltpu.sync_copy(p1_hbm.at[pl.ds(off, DCH)], p1v)
        pltpu.sync_copy(w0_hbm.at[pl.ds(off, DCH)], w0v)
        pltpu.sync_copy(w1_hbm.at[pl.ds(off, DCH)], w1v)
        c1 = pltpu.async_copy(xbuf, xg_hbm.at[p0v], s1)
        c2 = pltpu.async_copy(xbuf, xg_hbm.at[p1v], s2)
        c3 = pltpu.async_copy(w0v, ws_hbm.at[p0v], s3)
        c4 = pltpu.async_copy(w1v, ws_hbm.at[p1v], s4)
        c1.wait()
        c2.wait()
        c3.wait()
        c4.wait()
        return carry

    lax.fori_loop(0, TPW // DCH, body, 0)


@functools.partial(
    pl.kernel, mesh=_MESH,
    out_type=jax.ShapeDtypeStruct((T, DM), jnp.float32),
    scratch_types=[
        pltpu.VMEM((CCH, DM), jnp.float32),
        pltpu.VMEM((CCH, DM), jnp.float32),
        pltpu.VMEM((CCH,), jnp.int32),
        pltpu.VMEM((CCH,), jnp.int32),
        pltpu.SemaphoreType.DMA,
        pltpu.SemaphoreType.DMA,
    ],
)
def _sc_combine(og_hbm, p0_hbm, p1_hbm, y_hbm, bufa, bufb, p0v, p1v, sa, sb):
    wid = lax.axis_index("s") * 2 + lax.axis_index("c")
    base = wid * TPW

    def body(i, carry):
        off = base + i * CCH
        pltpu.sync_copy(p0_hbm.at[pl.ds(off, CCH)], p0v)
        pltpu.sync_copy(p1_hbm.at[pl.ds(off, CCH)], p1v)
        ca = pltpu.async_copy(og_hbm.at[p0v], bufa, sa)
        cb = pltpu.async_copy(og_hbm.at[p1v], bufb, sb)
        ca.wait()
        cb.wait()

        def add_row(j, c2):
            for k in range(DM // 16):
                sl = pl.ds(k * 16, 16)
                bufa[j, sl] = bufa[j, sl] + bufb[j, sl]
            return c2

        lax.fori_loop(0, CCH, add_row, 0)
        pltpu.sync_copy(bufa, y_hbm.at[pl.ds(off, CCH)])
        return carry

    lax.fori_loop(0, TPW // CCH, body, 0)


def kernel(x, Wg, bg, W_gate, b_gate, W_up, b_up, W_down, b_down):
    bs, slen, dim = x.shape
    xf = x.reshape(-1, dim)
    w0, w1, pos0, pos1, block_expert = _route(xf, Wg, bg)
    xg, w_slot = _sc_dispatch(xf, pos0, pos1, w0, w1)
    og = _ffn(xg, W_gate, b_gate, W_up, b_up, W_down, b_down,
              w_slot, block_expert)
    y = _sc_combine(og, pos0, pos1)
    return y.reshape(bs, slen, dim)


# speedup vs baseline: 1.0476x; 1.0476x over previous
import functools

import jax
import jax.numpy as jnp
from jax import lax
from jax.experimental import pallas as pl
from jax.experimental.pallas import tpu as pltpu
from jax.experimental.pallas import tpu_sc as plsc

DM = 1024
DF = 2048
NE = 8
TOPK = 2
T = 8192
A = T * TOPK
BLK = 512
NG = A // BLK + NE
S = NG * BLK
TB = 512


NTB = T // TB


def _route_body(x_ref, wg_ref, bg_ref,
                w0_ref, w1_ref, p0_ref, p1_ref, be_ref,
                e0s, e1s, w0s, w1s, cnt, cnt2):
    p = pl.program_id(0)
    g = pl.program_id(1)
    cols = lax.broadcasted_iota(jnp.int32, (TB, NE), 1)

    @pl.when(p == 0)
    def _phase0():
        logits = jnp.dot(x_ref[...], wg_ref[...],
                         preferred_element_type=jnp.float32) + bg_ref[0]
        i0 = jnp.argmax(logits, axis=1).astype(jnp.int32)
        m0 = jnp.max(logits, axis=1)
        masked = jnp.where(cols == i0[:, None], -jnp.inf, logits)
        i1 = jnp.argmax(masked, axis=1).astype(jnp.int32)
        m1 = jnp.max(masked, axis=1)
        w0 = jax.nn.sigmoid(m0 - m1)
        e0s[g, :] = i0
        e1s[g, :] = i1
        w0s[g, :] = w0
        w1s[g, :] = 1.0 - w0
        oh = ((cols == i0[:, None]).astype(jnp.float32)
              + (cols == i1[:, None]).astype(jnp.float32))
        colsum = jnp.sum(oh, axis=0, keepdims=True)
        prev = jnp.where(g == 0, jnp.zeros_like(cnt[...]), cnt[...])
        cnt[...] = prev + colsum

    @pl.when(p == 1)
    def _phase1():
        tot_i = cnt[...].astype(jnp.int32)
        bpe = (tot_i + BLK - 1) >> 9
        iu = lax.broadcasted_iota(jnp.int32, (NE, NE), 0)
        ju = lax.broadcasted_iota(jnp.int32, (NE, NE), 1)
        um = (iu < ju).astype(jnp.float32)
        bf = jnp.dot(bpe.astype(jnp.float32), um,
                     preferred_element_type=jnp.float32)
        prev2 = jnp.where(g == 0, jnp.zeros_like(cnt2[...]), cnt2[...])
        i0 = e0s[g, :]
        i1 = e1s[g, :]
        oh0 = (cols == i0[:, None]).astype(jnp.float32)
        oh1 = (cols == i1[:, None]).astype(jnp.float32)
        ohs = oh0 + oh1
        ri = lax.broadcasted_iota(jnp.int32, (TB, TB), 0)
        ci = lax.broadcasted_iota(jnp.int32, (TB, TB), 1)
        stril = (ci < ri).astype(jnp.bfloat16)
        cumbt = jnp.dot(stril, ohs.astype(jnp.bfloat16),
                        preferred_element_type=jnp.float32)
        base = prev2 + cumbt
        r0 = jnp.sum(base * oh0, axis=1).astype(jnp.int32)
        r1 = jnp.sum(base * oh1, axis=1).astype(jnp.int32)
        bfb = jnp.broadcast_to(bf, (TB, NE))
        bf0 = jnp.sum(bfb * oh0, axis=1).astype(jnp.int32)
        bf1 = jnp.sum(bfb * oh1, axis=1).astype(jnp.int32)
        p0_ref[...] = ((bf0 + (r0 >> 9)) << 9) + (r0 & (BLK - 1))
        p1_ref[...] = ((bf1 + (r1 >> 9)) << 9) + (r1 & (BLK - 1))
        w0_ref[...] = w0s[g, :]
        w1_ref[...] = w1s[g, :]
        cnt2[...] = prev2 + jnp.sum(ohs, axis=0, keepdims=True)
        bidx = lax.broadcasted_iota(jnp.int32, (NG, NE), 0)
        bfg = jnp.broadcast_to(bf, (NG, NE)).astype(jnp.int32)
        be_ref[...] = jnp.clip(
            jnp.sum((bidx >= bfg).astype(jnp.int32), axis=1) - 1, 0, NE - 1)


def _route(xf, Wg, bg):
    return pl.pallas_call(
        _route_body,
        grid=(2, NTB),
        in_specs=[
            pl.BlockSpec((TB, DM), lambda p, g: (g * (1 - p), 0)),
            pl.BlockSpec((DM, NE), lambda p, g: (0, 0)),
            pl.BlockSpec((1, NE), lambda p, g: (0, 0)),
        ],
        out_specs=[
            pl.BlockSpec((TB,), lambda p, g: (g,)),
            pl.BlockSpec((TB,), lambda p, g: (g,)),
            pl.BlockSpec((TB,), lambda p, g: (g,)),
            pl.BlockSpec((TB,), lambda p, g: (g,)),
            pl.BlockSpec((NG,), lambda p, g: (0,)),
        ],
        out_shape=[
            jax.ShapeDtypeStruct((T,), jnp.float32),
            jax.ShapeDtypeStruct((T,), jnp.float32),
            jax.ShapeDtypeStruct((T,), jnp.int32),
            jax.ShapeDtypeStruct((T,), jnp.int32),
            jax.ShapeDtypeStruct((NG,), jnp.int32),
        ],
        scratch_shapes=[
            pltpu.VMEM((NTB, TB), jnp.int32),
            pltpu.VMEM((NTB, TB), jnp.int32),
            pltpu.VMEM((NTB, TB), jnp.float32),
            pltpu.VMEM((NTB, TB), jnp.float32),
            pltpu.VMEM((1, NE), jnp.float32),
            pltpu.VMEM((1, NE), jnp.float32),
        ],
    )(xf, Wg, bg.reshape(1, NE))


def _ffn_body(be_ref, xg_ref, wg_ref, bg_ref, wu_ref, bu_ref, wd_ref, bd_ref,
              ws_ref, og_ref):
    xb = xg_ref[...].astype(jnp.bfloat16)
    g = jnp.dot(xb, wg_ref[0], preferred_element_type=jnp.float32) + bg_ref[0]
    u = jnp.dot(xb, wu_ref[0], preferred_element_type=jnp.float32) + bu_ref[0]
    t = g * u
    h = (t * jax.nn.sigmoid(t)).astype(jnp.bfloat16)
    o = jnp.dot(h, wd_ref[0], preferred_element_type=jnp.float32) + bd_ref[0]
    og_ref[...] = o * ws_ref[...]


def _ffn(xg, W_gate, b_gate, W_up, b_up, W_down, b_down, w_slot, block_expert):
    grid_spec = pltpu.PrefetchScalarGridSpec(
        num_scalar_prefetch=1,
        grid=(NG,),
        in_specs=[
            pl.BlockSpec((BLK, DM), lambda g, be: (g, 0)),
            pl.BlockSpec((1, DM, DF), lambda g, be: (be[g], 0, 0)),
            pl.BlockSpec((1, 1, DF), lambda g, be: (be[g], 0, 0)),
            pl.BlockSpec((1, DM, DF), lambda g, be: (be[g], 0, 0)),
            pl.BlockSpec((1, 1, DF), lambda g, be: (be[g], 0, 0)),
            pl.BlockSpec((1, DF, DM), lambda g, be: (be[g], 0, 0)),
            pl.BlockSpec((1, 1, DM), lambda g, be: (be[g], 0, 0)),
            pl.BlockSpec((BLK, 1), lambda g, be: (g, 0)),
        ],
        out_specs=pl.BlockSpec((BLK, DM), lambda g, be: (g, 0)),
    )
    return pl.pallas_call(
        _ffn_body,
        grid_spec=grid_spec,
        out_shape=jax.ShapeDtypeStruct((S, DM), jnp.float32),
    )(block_expert, xg, W_gate, b_gate.reshape(NE, 1, DF), W_up,
      b_up.reshape(NE, 1, DF), W_down, b_down.reshape(NE, 1, DM),
      w_slot.reshape(S, 1))


NW = 32
TPW = T // NW
DCH = 32
NDC = TPW // DCH
CCH = 16
NCC = TPW // CCH

_MESH = plsc.VectorSubcoreMesh(core_axis_name="c", subcore_axis_name="s")


@functools.partial(
    pl.kernel, mesh=_MESH,
    out_type=[jax.ShapeDtypeStruct((S, DM), jnp.float32),
              jax.ShapeDtypeStruct((S,), jnp.float32)],
    scratch_types=[
        pltpu.VMEM((2, DCH, DM), jnp.float32),
        pltpu.VMEM((NDC, DCH), jnp.int32),
        pltpu.VMEM((NDC, DCH), jnp.int32),
        pltpu.VMEM((NDC, DCH), jnp.float32),
        pltpu.VMEM((NDC, DCH), jnp.float32),
        pltpu.SemaphoreType.DMA,
        pltpu.SemaphoreType.DMA,
    ],
)
def _sc_dispatch(x_hbm, p0_hbm, p1_hbm, w0_hbm, w1_hbm, xg_hbm, ws_hbm,
                 xbuf, p0v, p1v, w0v, w1v, sx, ss):
    wid = lax.axis_index("s") * 2 + lax.axis_index("c")
    base = wid * TPW
    pltpu.sync_copy(p0_hbm.at[wid], p0v)
    pltpu.sync_copy(p1_hbm.at[wid], p1v)
    pltpu.sync_copy(w0_hbm.at[wid], w0v)
    pltpu.sync_copy(w1_hbm.at[wid], w1v)
    pltpu.async_copy(x_hbm.at[pl.ds(base, DCH)], xbuf.at[0], sx)

    def outer(io, carry):
        for b in range(2):
            i = io * 2 + b
            pltpu.make_async_copy(
                x_hbm.at[pl.ds(0, DCH)], xbuf.at[b], sx).wait()

            @pl.when(i + 1 < NDC)
            def _():
                pltpu.async_copy(
                    x_hbm.at[pl.ds(base + (i + 1) * DCH, DCH)],
                    xbuf.at[1 - b], sx)

            c1 = pltpu.async_copy(xbuf.at[b], xg_hbm.at[p0v.at[i]], ss)
            c2 = pltpu.async_copy(xbuf.at[b], xg_hbm.at[p1v.at[i]], ss)
            c3 = pltpu.async_copy(w0v.at[i], ws_hbm.at[p0v.at[i]], ss)
            c4 = pltpu.async_copy(w1v.at[i], ws_hbm.at[p1v.at[i]], ss)
            c1.wait()
            c2.wait()
            c3.wait()
            c4.wait()
        return carry

    lax.fori_loop(0, NDC // 2, outer, 0)


@functools.partial(
    pl.kernel, mesh=_MESH,
    out_type=jax.ShapeDtypeStruct((T, DM), jnp.float32),
    scratch_types=[
        pltpu.VMEM((2, CCH, DM), jnp.float32),
        pltpu.VMEM((2, CCH, DM), jnp.float32),
        pltpu.VMEM((NCC, CCH), jnp.int32),
        pltpu.VMEM((NCC, CCH), jnp.int32),
        pltpu.SemaphoreType.DMA,
    ],
)
def _sc_combine(og_hbm, p0_hbm, p1_hbm, y_hbm, bufa, bufb, p0v, p1v, sg):
    wid = lax.axis_index("s") * 2 + lax.axis_index("c")
    base = wid * TPW
    pltpu.sync_copy(p0_hbm.at[wid], p0v)
    pltpu.sync_copy(p1_hbm.at[wid], p1v)
    pltpu.async_copy(og_hbm.at[p0v.at[0]], bufa.at[0], sg)
    pltpu.async_copy(og_hbm.at[p1v.at[0]], bufb.at[0], sg)

    def outer(io, carry):
        for b in range(2):
            i = io * 2 + b
            pltpu.make_async_copy(
                og_hbm.at[pl.ds(0, CCH)], bufa.at[b], sg).wait()
            pltpu.make_async_copy(
                og_hbm.at[pl.ds(0, CCH)], bufb.at[b], sg).wait()

            @pl.when(i + 1 < NCC)
            def _():
                pltpu.async_copy(og_hbm.at[p0v.at[i + 1]], bufa.at[1 - b], sg)
                pltpu.async_copy(og_hbm.at[p1v.at[i + 1]], bufb.at[1 - b], sg)

            def add_row(j, c2):
                for k in range(DM // 16):
                    sl = pl.ds(k * 16, 16)
                    bufa[b, j, sl] = bufa[b, j, sl] + bufb[b, j, sl]
                return c2

            lax.fori_loop(0, CCH, add_row, 0)
            pltpu.sync_copy(bufa.at[b],
                            y_hbm.at[pl.ds(base + i * CCH, CCH)])
        return carry

    lax.fori_loop(0, NCC // 2, outer, 0)


def kernel(x, Wg, bg, W_gate, b_gate, W_up, b_up, W_down, b_down):
    bs, slen, dim = x.shape
    xf = x.reshape(-1, dim)
    w0, w1, pos0, pos1, block_expert = _route(xf, Wg, bg)
    xg, w_slot = _sc_dispatch(xf,
                              pos0.reshape(NW, NDC, DCH),
                              pos1.reshape(NW, NDC, DCH),
                              w0.reshape(NW, NDC, DCH),
                              w1.reshape(NW, NDC, DCH))
    og = _ffn(xg, W_gate.astype(jnp.bfloat16), b_gate,
              W_up.astype(jnp.bfloat16), b_up,
              W_down.astype(jnp.bfloat16), b_down, w_slot, block_expert)
    y = _sc_combine(og, pos0.reshape(NW, NCC, CCH), pos1.reshape(NW, NCC, CCH))
    return y.reshape(bs, slen, dim)


# speedup vs baseline: 1.1158x; 1.0651x over previous
import functools

import jax
import jax.numpy as jnp
from jax import lax
from jax.experimental import pallas as pl
from jax.experimental.pallas import tpu as pltpu
from jax.experimental.pallas import tpu_sc as plsc

DM = 1024
DF = 2048
NE = 8
TOPK = 2
T = 8192
A = T * TOPK
BLK = 512
NG = A // BLK + NE
S = NG * BLK
TB = 512


NTB = T // TB


def _route_body(x_ref, wg_ref, bg_ref,
                w0_ref, w1_ref, p0_ref, p1_ref, be_ref,
                e0s, e1s, w0s, w1s, cnt, cnt2):
    p = pl.program_id(0)
    g = pl.program_id(1)
    cols = lax.broadcasted_iota(jnp.int32, (TB, NE), 1)

    @pl.when(p == 0)
    def _phase0():
        logits = jnp.dot(x_ref[...], wg_ref[...],
                         preferred_element_type=jnp.float32) + bg_ref[0]
        i0 = jnp.argmax(logits, axis=1).astype(jnp.int32)
        m0 = jnp.max(logits, axis=1)
        masked = jnp.where(cols == i0[:, None], -jnp.inf, logits)
        i1 = jnp.argmax(masked, axis=1).astype(jnp.int32)
        m1 = jnp.max(masked, axis=1)
        w0 = jax.nn.sigmoid(m0 - m1)
        e0s[g, :] = i0
        e1s[g, :] = i1
        w0s[g, :] = w0
        w1s[g, :] = 1.0 - w0
        oh = ((cols == i0[:, None]).astype(jnp.float32)
              + (cols == i1[:, None]).astype(jnp.float32))
        colsum = jnp.sum(oh, axis=0, keepdims=True)
        prev = jnp.where(g == 0, jnp.zeros_like(cnt[...]), cnt[...])
        cnt[...] = prev + colsum

    @pl.when(p == 1)
    def _phase1():
        tot_i = cnt[...].astype(jnp.int32)
        bpe = (tot_i + BLK - 1) >> 9
        iu = lax.broadcasted_iota(jnp.int32, (NE, NE), 0)
        ju = lax.broadcasted_iota(jnp.int32, (NE, NE), 1)
        um = (iu < ju).astype(jnp.float32)
        bf = jnp.dot(bpe.astype(jnp.float32), um,
                     preferred_element_type=jnp.float32)
        prev2 = jnp.where(g == 0, jnp.zeros_like(cnt2[...]), cnt2[...])
        i0 = e0s[g, :]
        i1 = e1s[g, :]
        oh0 = (cols == i0[:, None]).astype(jnp.float32)
        oh1 = (cols == i1[:, None]).astype(jnp.float32)
        ohs = oh0 + oh1
        ri = lax.broadcasted_iota(jnp.int32, (TB, TB), 0)
        ci = lax.broadcasted_iota(jnp.int32, (TB, TB), 1)
        stril = (ci < ri).astype(jnp.bfloat16)
        cumbt = jnp.dot(stril, ohs.astype(jnp.bfloat16),
                        preferred_element_type=jnp.float32)
        base = prev2 + cumbt
        r0 = jnp.sum(base * oh0, axis=1).astype(jnp.int32)
        r1 = jnp.sum(base * oh1, axis=1).astype(jnp.int32)
        bfb = jnp.broadcast_to(bf, (TB, NE))
        bf0 = jnp.sum(bfb * oh0, axis=1).astype(jnp.int32)
        bf1 = jnp.sum(bfb * oh1, axis=1).astype(jnp.int32)
        p0_ref[...] = ((bf0 + (r0 >> 9)) << 9) + (r0 & (BLK - 1))
        p1_ref[...] = ((bf1 + (r1 >> 9)) << 9) + (r1 & (BLK - 1))
        w0_ref[...] = w0s[g, :]
        w1_ref[...] = w1s[g, :]
        cnt2[...] = prev2 + jnp.sum(ohs, axis=0, keepdims=True)
        bidx = lax.broadcasted_iota(jnp.int32, (NG, NE), 0)
        bfg = jnp.broadcast_to(bf, (NG, NE)).astype(jnp.int32)
        be_ref[...] = jnp.clip(
            jnp.sum((bidx >= bfg).astype(jnp.int32), axis=1) - 1, 0, NE - 1)


def _route(xf, Wg, bg):
    return pl.pallas_call(
        _route_body,
        grid=(2, NTB),
        in_specs=[
            pl.BlockSpec((TB, DM), lambda p, g: (g * (1 - p), 0)),
            pl.BlockSpec((DM, NE), lambda p, g: (0, 0)),
            pl.BlockSpec((1, NE), lambda p, g: (0, 0)),
        ],
        out_specs=[
            pl.BlockSpec((TB,), lambda p, g: (g,)),
            pl.BlockSpec((TB,), lambda p, g: (g,)),
            pl.BlockSpec((TB,), lambda p, g: (g,)),
            pl.BlockSpec((TB,), lambda p, g: (g,)),
            pl.BlockSpec((NG,), lambda p, g: (0,)),
        ],
        out_shape=[
            jax.ShapeDtypeStruct((T,), jnp.float32),
            jax.ShapeDtypeStruct((T,), jnp.float32),
            jax.ShapeDtypeStruct((T,), jnp.int32),
            jax.ShapeDtypeStruct((T,), jnp.int32),
            jax.ShapeDtypeStruct((NG,), jnp.int32),
        ],
        scratch_shapes=[
            pltpu.VMEM((NTB, TB), jnp.int32),
            pltpu.VMEM((NTB, TB), jnp.int32),
            pltpu.VMEM((NTB, TB), jnp.float32),
            pltpu.VMEM((NTB, TB), jnp.float32),
            pltpu.VMEM((1, NE), jnp.float32),
            pltpu.VMEM((1, NE), jnp.float32),
        ],
    )(xf, Wg, bg.reshape(1, NE))


def _ffn_body(be_ref, xg_ref, wg_ref, bg_ref, wu_ref, bu_ref, wd_ref, bd_ref,
              og_ref):
    xb = xg_ref[...].astype(jnp.bfloat16)
    g = jnp.dot(xb, wg_ref[0], preferred_element_type=jnp.float32) + bg_ref[0]
    u = jnp.dot(xb, wu_ref[0], preferred_element_type=jnp.float32) + bu_ref[0]
    t = g * u
    h = (t * jax.nn.sigmoid(t)).astype(jnp.bfloat16)
    og_ref[...] = (jnp.dot(h, wd_ref[0], preferred_element_type=jnp.float32)
                   + bd_ref[0])


def _ffn(xg, W_gate, b_gate, W_up, b_up, W_down, b_down, block_expert):
    grid_spec = pltpu.PrefetchScalarGridSpec(
        num_scalar_prefetch=1,
        grid=(NG,),
        in_specs=[
            pl.BlockSpec((BLK, DM), lambda g, be: (g, 0)),
            pl.BlockSpec((1, DM, DF), lambda g, be: (be[g], 0, 0)),
            pl.BlockSpec((1, 1, DF), lambda g, be: (be[g], 0, 0)),
            pl.BlockSpec((1, DM, DF), lambda g, be: (be[g], 0, 0)),
            pl.BlockSpec((1, 1, DF), lambda g, be: (be[g], 0, 0)),
            pl.BlockSpec((1, DF, DM), lambda g, be: (be[g], 0, 0)),
            pl.BlockSpec((1, 1, DM), lambda g, be: (be[g], 0, 0)),
        ],
        out_specs=pl.BlockSpec((BLK, DM), lambda g, be: (g, 0)),
    )
    return pl.pallas_call(
        _ffn_body,
        grid_spec=grid_spec,
        out_shape=jax.ShapeDtypeStruct((S, DM), jnp.float32),
    )(block_expert, xg, W_gate, b_gate.reshape(NE, 1, DF), W_up,
      b_up.reshape(NE, 1, DF), W_down, b_down.reshape(NE, 1, DM))


NW = 32
TPW = T // NW
DCH = 16
NDC = TPW // DCH
CCH = 8
NCC = TPW // CCH

_MESH = plsc.VectorSubcoreMesh(core_axis_name="c", subcore_axis_name="s")


@functools.partial(
    pl.kernel, mesh=_MESH,
    out_type=jax.ShapeDtypeStruct((S, DM), jnp.float32),
    scratch_types=[
        pltpu.VMEM((2, DCH, DM), jnp.float32),
        pltpu.VMEM((NDC, DCH), jnp.int32),
        pltpu.VMEM((NDC, DCH), jnp.int32),
        pltpu.SemaphoreType.DMA,
        pltpu.SemaphoreType.DMA,
    ],
)
def _sc_dispatch(x_hbm, p0_hbm, p1_hbm, xg_hbm, xbuf, p0v, p1v, sx, ss):
    wid = lax.axis_index("s") * 2 + lax.axis_index("c")
    base = wid * TPW
    pltpu.sync_copy(p0_hbm.at[wid], p0v)
    pltpu.sync_copy(p1_hbm.at[wid], p1v)
    pltpu.async_copy(x_hbm.at[pl.ds(base, DCH)], xbuf.at[0], sx)

    def outer(io, carry):
        for b in range(2):
            i = io * 2 + b
            pltpu.make_async_copy(
                x_hbm.at[pl.ds(0, DCH)], xbuf.at[b], sx).wait()

            @pl.when(i + 1 < NDC)
            def _():
                pltpu.async_copy(
                    x_hbm.at[pl.ds(base + (i + 1) * DCH, DCH)],
                    xbuf.at[1 - b], sx)

            c1 = pltpu.async_copy(xbuf.at[b], xg_hbm.at[p0v.at[i]], ss)
            c2 = pltpu.async_copy(xbuf.at[b], xg_hbm.at[p1v.at[i]], ss)
            c1.wait()
            c2.wait()
        return carry

    lax.fori_loop(0, NDC // 2, outer, 0)


@functools.partial(
    pl.kernel, mesh=_MESH,
    out_type=jax.ShapeDtypeStruct((T, DM), jnp.float32),
    scratch_types=[
        pltpu.VMEM((2, CCH, DM), jnp.float32),
        pltpu.VMEM((2, CCH, DM), jnp.float32),
        pltpu.VMEM((NCC, CCH), jnp.int32),
        pltpu.VMEM((NCC, CCH), jnp.int32),
        pltpu.VMEM((NCC, CCH, 16), jnp.float32),
        pltpu.VMEM((NCC, CCH, 16), jnp.float32),
        pltpu.SemaphoreType.DMA,
    ],
)
def _sc_combine(og_hbm, p0_hbm, p1_hbm, w0_hbm, w1_hbm, y_hbm,
                bufa, bufb, p0v, p1v, w0v, w1v, sg):
    wid = lax.axis_index("s") * 2 + lax.axis_index("c")
    base = wid * TPW
    pltpu.sync_copy(p0_hbm.at[wid], p0v)
    pltpu.sync_copy(p1_hbm.at[wid], p1v)
    pltpu.sync_copy(w0_hbm.at[wid], w0v)
    pltpu.sync_copy(w1_hbm.at[wid], w1v)
    pltpu.async_copy(og_hbm.at[p0v.at[0]], bufa.at[0], sg)
    pltpu.async_copy(og_hbm.at[p1v.at[0]], bufb.at[0], sg)

    def outer(io, carry):
        for b in range(2):
            i = io * 2 + b
            pltpu.make_async_copy(
                og_hbm.at[pl.ds(0, CCH)], bufa.at[b], sg).wait()
            pltpu.make_async_copy(
                og_hbm.at[pl.ds(0, CCH)], bufb.at[b], sg).wait()

            @pl.when(i + 1 < NCC)
            def _():
                pltpu.async_copy(og_hbm.at[p0v.at[i + 1]], bufa.at[1 - b], sg)
                pltpu.async_copy(og_hbm.at[p1v.at[i + 1]], bufb.at[1 - b], sg)

            def add_row(j, c2):
                wa = w0v[i, j, :]
                wb = w1v[i, j, :]
                for k in range(DM // 16):
                    sl = pl.ds(k * 16, 16)
                    bufa[b, j, sl] = bufa[b, j, sl] * wa + bufb[b, j, sl] * wb
                return c2

            lax.fori_loop(0, CCH, add_row, 0)
            pltpu.sync_copy(bufa.at[b],
                            y_hbm.at[pl.ds(base + i * CCH, CCH)])
        return carry

    lax.fori_loop(0, NCC // 2, outer, 0)


def kernel(x, Wg, bg, W_gate, b_gate, W_up, b_up, W_down, b_down):
    bs, slen, dim = x.shape
    xf = x.reshape(-1, dim)
    w0, w1, pos0, pos1, block_expert = _route(xf, Wg, bg)
    xg = _sc_dispatch(xf,
                      pos0.reshape(NW, NDC, DCH),
                      pos1.reshape(NW, NDC, DCH))
    og = _ffn(xg, W_gate.astype(jnp.bfloat16), b_gate,
              W_up.astype(jnp.bfloat16), b_up,
              W_down.astype(jnp.bfloat16), b_down, block_expert)
    w0e = jnp.broadcast_to(w0[:, None], (T, 16))
    w1e = jnp.broadcast_to(w1[:, None], (T, 16))
    y = _sc_combine(og, pos0.reshape(NW, NCC, CCH), pos1.reshape(NW, NCC, CCH),
                    w0e.reshape(NW, NCC, CCH, 16),
                    w1e.reshape(NW, NCC, CCH, 16))
    return y.reshape(bs, slen, dim)
